# baseline (speedup 1.0000x reference)
import functools

import jax
import jax.numpy as jnp
from jax.experimental import pallas as pl
from jax.experimental.pallas import tpu as pltpu

LARGE_NUMBER = 100000000.0
_CHUNK = 16


def _decoder_body(S, B, H, att, qdim,
                  tin_ref, wdt_ref, w0t_ref, w1t_ref, w2t_ref, wpt_ref,
                  v_ref, g_ref,
                  tour_ref, lp_ref, ent_ref,
                  e_ref, mask_ref, scores_ref, rnew_ref,
                  qsum_ref, eq_ref, idx_vref, idx_sref, dma_sem):
    f32 = jnp.float32
    nchunk = S // _CHUNK

    def fill_e(c, _):
        x = tin_ref[pl.ds(c * _CHUNK, _CHUNK)]
        x2 = x.reshape(_CHUNK * B, H)
        e2 = jnp.dot(x2, wdt_ref[:], preferred_element_type=f32)
        e_ref[pl.ds(c * _CHUNK, _CHUNK)] = e2.reshape(_CHUNK, B, att)
        return 0

    jax.lax.fori_loop(0, nchunk, fill_e, 0)

    mask_ref[:] = jnp.zeros((S, B), dtype=f32)
    rnew_ref[:] = jnp.zeros((B, H), dtype=f32)
    qsum_ref[:] = jnp.zeros((4, B, qdim), dtype=f32)
    eq_ref[:] = jnp.zeros((B, att), dtype=f32)

    iota_s = jax.lax.broadcasted_iota(jnp.int32, (S, B), 0)
    v_row = v_ref[:]

    def step(t, carry):
        lp, ent = carry
        eq = eq_ref[:]

        r_prev = rnew_ref[:]
        p1 = jnp.dot(r_prev, w1t_ref[:], preferred_element_type=f32)
        s1 = jax.lax.rem(t + 1, 4)
        qsum_ref[pl.ds(s1, 1)] = qsum_ref[pl.ds(s1, 1)] + p1[None]
        p0 = jnp.dot(r_prev, w0t_ref[:], preferred_element_type=f32)
        s0 = jax.lax.rem(t + 2, 4)
        qsum_ref[pl.ds(s0, 1)] = qsum_ref[pl.ds(s0, 1)] + p0[None]
        sz = jax.lax.rem(t, 4)
        qsum_ref[pl.ds(sz, 1)] = jnp.zeros((1, B, qdim), dtype=f32)

        def score_chunk(c, _):
            ec = e_ref[pl.ds(c * _CHUNK, _CHUNK)]
            x = jnp.tanh(ec + eq[None, :, :])
            scores_ref[pl.ds(c * _CHUNK, _CHUNK)] = jnp.sum(
                v_row[None, :, :] * x, axis=-1)
            return 0

        jax.lax.fori_loop(0, nchunk, score_chunk, 0)

        scores = 10.0 * jnp.tanh(scores_ref[:])
        masked = jnp.clip(scores - LARGE_NUMBER * mask_ref[:],
                          -LARGE_NUMBER, LARGE_NUMBER)

        z = masked + g_ref[t]
        zmax = jnp.max(z, axis=0)
        idx = jnp.min(jnp.where(z == zmax[None, :], iota_s, S), axis=0)
        oh = (iota_s == idx[None, :]).astype(f32)

        xmax = jnp.max(masked, axis=0)
        sh = masked - xmax[None, :]
        logz = jnp.log(jnp.sum(jnp.exp(sh), axis=0))
        chosen = jnp.sum(oh * sh, axis=0)
        lp = lp + (chosen - logz)[None, :]
        logp = sh - logz[None, :]
        probs = jnp.exp(logp)
        ent = ent - jnp.sum(probs * logp, axis=0)[None, :]

        tour_ref[pl.ds(t, 1)] = idx[None, :]
        mask_ref[:] = mask_ref[:] + oh

        idx_vref[:] = idx[None, :]
        cp = pltpu.make_async_copy(idx_vref, idx_sref, dma_sem)
        cp.start()
        cp.wait()

        def gather_b(b, _):
            iv = idx_sref[0, b]
            rnew_ref[pl.ds(b, 1), :] = tin_ref[iv, pl.ds(b, 1), :]
            return 0

        jax.lax.fori_loop(0, B, gather_b, 0)

        p2 = jnp.dot(rnew_ref[:], w2t_ref[:], preferred_element_type=f32)
        qn = qsum_ref[s1] + p2
        query = jnp.maximum(qn, 0.0)
        eq_ref[:] = jnp.dot(query, wpt_ref[:], preferred_element_type=f32)
        return (lp, ent)

    lp0 = jnp.zeros((1, B), dtype=f32)
    ent0 = jnp.zeros((1, B), dtype=f32)
    lp, ent = jax.lax.fori_loop(0, S, step, (lp0, ent0))
    lp_ref[:] = lp
    ent_ref[:] = ent


def kernel(inputs, W_dense, W_q0, W_q1, W_q2, Wp, v):
    B, S, H = inputs.shape
    att = W_dense.shape[0]
    qdim = W_q0.shape[0]
    f32 = jnp.float32

    tin = jnp.transpose(inputs, (1, 0, 2))
    wdt = W_dense.T
    w0t, w1t, w2t = W_q0.T, W_q1.T, W_q2.T
    wpt = Wp.T
    v_row = v.reshape(1, att)

    skey = jax.random.key(42)
    keys = jax.vmap(lambda t: jax.random.fold_in(skey, t))(jnp.arange(S))
    g = jax.vmap(lambda k: jax.random.gumbel(k, (B, S), f32))(keys)
    g = jnp.transpose(g, (0, 2, 1))

    body = functools.partial(_decoder_body, S, B, H, att, qdim)
    tour_steps, lp, ent = pl.pallas_call(
        body,
        out_shape=(
            jax.ShapeDtypeStruct((S, B), jnp.int32),
            jax.ShapeDtypeStruct((1, B), f32),
            jax.ShapeDtypeStruct((1, B), f32),
        ),
        scratch_shapes=[
            pltpu.VMEM((S, B, att), f32),
            pltpu.VMEM((S, B), f32),
            pltpu.VMEM((S, B), f32),
            pltpu.VMEM((B, H), f32),
            pltpu.VMEM((4, B, qdim), f32),
            pltpu.VMEM((B, att), f32),
            pltpu.VMEM((1, B), jnp.int32),
            pltpu.SMEM((1, B), jnp.int32),
            pltpu.SemaphoreType.DMA,
        ],
        compiler_params=pltpu.CompilerParams(
            vmem_limit_bytes=128 * 1024 * 1024),
    )(tin, wdt, w0t, w1t, w2t, wpt, v_row, g)

    steps_t = tour_steps.T
    tour = jnp.concatenate([steps_t, steps_t[:, :1]], axis=1)
    return (tour, lp[0], ent[0])


# speedup vs baseline: 1.1527x; 1.1527x over previous
import functools

import jax
import jax.numpy as jnp
from jax.experimental import pallas as pl
from jax.experimental.pallas import tpu as pltpu

LARGE_NUMBER = 100000000.0
_CHUNK = 16


def _decoder_body(S, B, H, att, qdim,
                  tin_ref, wdt_ref, w0t_ref, w1t_ref, w2t_ref, wpt_ref,
                  v_ref, g_ref,
                  tour_ref, lp_ref, ent_ref,
                  e_ref, mask_ref, scores_ref, rnew_ref,
                  qsum_ref, eq_ref, idx_vref, idx_sref, dma_sem):
    f32 = jnp.float32
    nchunk = S // _CHUNK

    def fill_e(c, _):
        x = tin_ref[pl.ds(c * _CHUNK, _CHUNK)]
        x2 = x.reshape(_CHUNK * B, H)
        e2 = jnp.dot(x2, wdt_ref[:], preferred_element_type=f32)
        e_ref[pl.ds(c * _CHUNK, _CHUNK)] = e2.reshape(_CHUNK, B, att)
        return 0

    jax.lax.fori_loop(0, nchunk, fill_e, 0)

    mask_ref[:] = jnp.zeros((S, B), dtype=f32)
    rnew_ref[:] = jnp.zeros((B, H), dtype=f32)
    qsum_ref[:] = jnp.zeros((4, B, qdim), dtype=f32)
    eq_ref[:] = jnp.zeros((B, att), dtype=f32)

    iota_s = jax.lax.broadcasted_iota(jnp.int32, (S, B), 0)
    v_row = v_ref[:]

    def step(t, carry):
        lp, ent = carry
        eq = eq_ref[:]

        r_prev = rnew_ref[:]
        p1 = jnp.dot(r_prev, w1t_ref[:], preferred_element_type=f32)
        s1 = jax.lax.rem(t + 1, 4)
        qsum_ref[pl.ds(s1, 1)] = qsum_ref[pl.ds(s1, 1)] + p1[None]
        p0 = jnp.dot(r_prev, w0t_ref[:], preferred_element_type=f32)
        s0 = jax.lax.rem(t + 2, 4)
        qsum_ref[pl.ds(s0, 1)] = qsum_ref[pl.ds(s0, 1)] + p0[None]
        sz = jax.lax.rem(t, 4)
        qsum_ref[pl.ds(sz, 1)] = jnp.zeros((1, B, qdim), dtype=f32)

        for c in range(nchunk):
            ec = e_ref[pl.ds(c * _CHUNK, _CHUNK)]
            x = jnp.tanh(ec + eq[None, :, :])
            scores_ref[pl.ds(c * _CHUNK, _CHUNK)] = jnp.sum(
                v_row[None, :, :] * x, axis=-1)

        scores = 10.0 * jnp.tanh(scores_ref[:])
        masked = jnp.clip(scores - LARGE_NUMBER * mask_ref[:],
                          -LARGE_NUMBER, LARGE_NUMBER)

        z = masked + g_ref[t]
        zmax = jnp.max(z, axis=0)
        idx = jnp.min(jnp.where(z == zmax[None, :], iota_s, S), axis=0)
        oh = (iota_s == idx[None, :]).astype(f32)

        xmax = jnp.max(masked, axis=0)
        sh = masked - xmax[None, :]
        logz = jnp.log(jnp.sum(jnp.exp(sh), axis=0))
        chosen = jnp.sum(oh * sh, axis=0)
        lp = lp + (chosen - logz)[None, :]
        logp = sh - logz[None, :]
        probs = jnp.exp(logp)
        ent = ent - jnp.sum(probs * logp, axis=0)[None, :]

        tour_ref[pl.ds(t, 1)] = idx[None, :]
        mask_ref[:] = mask_ref[:] + oh

        idx_vref[:] = idx[None, :]
        cp = pltpu.make_async_copy(idx_vref, idx_sref, dma_sem)
        cp.start()
        cp.wait()

        def gather_b(b, _):
            iv = idx_sref[0, b]
            rnew_ref[pl.ds(b, 1), :] = tin_ref[iv, pl.ds(b, 1), :]
            return 0

        jax.lax.fori_loop(0, B, gather_b, 0)

        p2 = jnp.dot(rnew_ref[:], w2t_ref[:], preferred_element_type=f32)
        qn = qsum_ref[s1] + p2
        query = jnp.maximum(qn, 0.0)
        eq_ref[:] = jnp.dot(query, wpt_ref[:], preferred_element_type=f32)
        return (lp, ent)

    lp0 = jnp.zeros((1, B), dtype=f32)
    ent0 = jnp.zeros((1, B), dtype=f32)
    lp, ent = jax.lax.fori_loop(0, S, step, (lp0, ent0))
    lp_ref[:] = lp
    ent_ref[:] = ent


def kernel(inputs, W_dense, W_q0, W_q1, W_q2, Wp, v):
    B, S, H = inputs.shape
    att = W_dense.shape[0]
    qdim = W_q0.shape[0]
    f32 = jnp.float32

    tin = jnp.transpose(inputs, (1, 0, 2))
    wdt = W_dense.T
    w0t, w1t, w2t = W_q0.T, W_q1.T, W_q2.T
    wpt = Wp.T
    v_row = v.reshape(1, att)

    skey = jax.random.key(42)
    keys = jax.vmap(lambda t: jax.random.fold_in(skey, t))(jnp.arange(S))
    g = jax.vmap(lambda k: jax.random.gumbel(k, (B, S), f32))(keys)
    g = jnp.transpose(g, (0, 2, 1))

    body = functools.partial(_decoder_body, S, B, H, att, qdim)
    tour_steps, lp, ent = pl.pallas_call(
        body,
        out_shape=(
            jax.ShapeDtypeStruct((S, B), jnp.int32),
            jax.ShapeDtypeStruct((1, B), f32),
            jax.ShapeDtypeStruct((1, B), f32),
        ),
        scratch_shapes=[
            pltpu.VMEM((S, B, att), f32),
            pltpu.VMEM((S, B), f32),
            pltpu.VMEM((S, B), f32),
            pltpu.VMEM((B, H), f32),
            pltpu.VMEM((4, B, qdim), f32),
            pltpu.VMEM((B, att), f32),
            pltpu.VMEM((1, B), jnp.int32),
            pltpu.SMEM((1, B), jnp.int32),
            pltpu.SemaphoreType.DMA,
        ],
        compiler_params=pltpu.CompilerParams(
            vmem_limit_bytes=128 * 1024 * 1024),
    )(tin, wdt, w0t, w1t, w2t, wpt, v_row, g)

    steps_t = tour_steps.T
    tour = jnp.concatenate([steps_t, steps_t[:, :1]], axis=1)
    return (tour, lp[0], ent[0])


# speedup vs baseline: 1.3546x; 1.1752x over previous
import functools

import jax
import jax.numpy as jnp
from jax.experimental import pallas as pl
from jax.experimental.pallas import tpu as pltpu

LARGE_NUMBER = 100000000.0
_CHUNK = 16


def _decoder_body(S, B, H, att, qdim,
                  tin_ref, wdt_ref, w0t_ref, w1t_ref, w2t_ref, wpt_ref,
                  v_ref, g_ref,
                  tour_ref, lp_ref, ent_ref,
                  e_ref, mask_ref, scores_ref, rnew_ref,
                  qsum_ref, eq_ref, idx_vref, idx_sref, dma_sem):
    f32 = jnp.float32
    nchunk = S // _CHUNK

    def fill_e(c, _):
        x = tin_ref[pl.ds(c * _CHUNK, _CHUNK)]
        x2 = x.reshape(_CHUNK * B, H)
        e2 = jnp.dot(x2, wdt_ref[:], preferred_element_type=f32)
        e_ref[pl.ds(c * _CHUNK, _CHUNK)] = e2.reshape(_CHUNK, B, att)
        return 0

    jax.lax.fori_loop(0, nchunk, fill_e, 0)

    mask_ref[:] = jnp.zeros((S, B), dtype=f32)
    rnew_ref[:] = jnp.zeros((B, H), dtype=f32)
    qsum_ref[:] = jnp.zeros((4, B, qdim), dtype=f32)
    eq_ref[:] = jnp.zeros((B, att), dtype=f32)

    iota_s = jax.lax.broadcasted_iota(jnp.int32, (S, B), 0)
    v_row = v_ref[:]

    def step(t, carry):
        lp, ent = carry
        eq = eq_ref[:]

        r_prev = rnew_ref[:]
        p1 = jnp.dot(r_prev, w1t_ref[:], preferred_element_type=f32)
        s1 = jax.lax.rem(t + 1, 4)
        qsum_ref[pl.ds(s1, 1)] = qsum_ref[pl.ds(s1, 1)] + p1[None]
        p0 = jnp.dot(r_prev, w0t_ref[:], preferred_element_type=f32)
        s0 = jax.lax.rem(t + 2, 4)
        qsum_ref[pl.ds(s0, 1)] = qsum_ref[pl.ds(s0, 1)] + p0[None]
        sz = jax.lax.rem(t, 4)
        qsum_ref[pl.ds(sz, 1)] = jnp.zeros((1, B, qdim), dtype=f32)

        for c in range(nchunk):
            ec = e_ref[pl.ds(c * _CHUNK, _CHUNK)]
            x = jnp.tanh(ec + eq[None, :, :])
            scores_ref[pl.ds(c * _CHUNK, _CHUNK)] = jnp.sum(
                v_row[None, :, :] * x, axis=-1)

        scores = 10.0 * jnp.tanh(scores_ref[:])
        masked = jnp.clip(scores - LARGE_NUMBER * mask_ref[:],
                          -LARGE_NUMBER, LARGE_NUMBER)

        z = masked + g_ref[t]
        zmax = jnp.max(z, axis=0)
        idx = jnp.min(jnp.where(z == zmax[None, :], iota_s, S), axis=0)
        oh = (iota_s == idx[None, :]).astype(f32)

        xmax = jnp.max(masked, axis=0)
        sh = masked - xmax[None, :]
        logz = jnp.log(jnp.sum(jnp.exp(sh), axis=0))
        chosen = jnp.sum(oh * sh, axis=0)
        lp = lp + (chosen - logz)[None, :]
        logp = sh - logz[None, :]
        probs = jnp.exp(logp)
        ent = ent - jnp.sum(probs * logp, axis=0)[None, :]

        tour_ref[pl.ds(t, 1)] = idx[None, :]
        mask_ref[:] = mask_ref[:] + oh

        idx_vref[:] = idx[None, :]
        cp = pltpu.make_async_copy(idx_vref, idx_sref, dma_sem)
        cp.start()
        cp.wait()

        for b in range(B):
            iv = idx_sref[0, b]
            rnew_ref[pl.ds(b, 1), :] = tin_ref[iv, pl.ds(b, 1), :]

        half = B // 2
        for c in range(2):
            rows = pl.ds(c * half, half)
            p2 = jnp.dot(rnew_ref[rows, :], w2t_ref[:],
                         preferred_element_type=f32)
            qn = qsum_ref[s1, rows, :] + p2
            query = jnp.maximum(qn, 0.0)
            eq_ref[rows, :] = jnp.dot(query, wpt_ref[:],
                                      preferred_element_type=f32)
        return (lp, ent)

    lp0 = jnp.zeros((1, B), dtype=f32)
    ent0 = jnp.zeros((1, B), dtype=f32)
    lp, ent = jax.lax.fori_loop(0, S, step, (lp0, ent0))
    lp_ref[:] = lp
    ent_ref[:] = ent


def kernel(inputs, W_dense, W_q0, W_q1, W_q2, Wp, v):
    B, S, H = inputs.shape
    att = W_dense.shape[0]
    qdim = W_q0.shape[0]
    f32 = jnp.float32

    tin = jnp.transpose(inputs, (1, 0, 2))
    wdt = W_dense.T
    w0t, w1t, w2t = W_q0.T, W_q1.T, W_q2.T
    wpt = Wp.T
    v_row = v.reshape(1, att)

    skey = jax.random.key(42)
    keys = jax.vmap(lambda t: jax.random.fold_in(skey, t))(jnp.arange(S))
    g = jax.vmap(lambda k: jax.random.gumbel(k, (B, S), f32))(keys)
    g = jnp.transpose(g, (0, 2, 1))

    body = functools.partial(_decoder_body, S, B, H, att, qdim)
    tour_steps, lp, ent = pl.pallas_call(
        body,
        out_shape=(
            jax.ShapeDtypeStruct((S, B), jnp.int32),
            jax.ShapeDtypeStruct((1, B), f32),
            jax.ShapeDtypeStruct((1, B), f32),
        ),
        scratch_shapes=[
            pltpu.VMEM((S, B, att), f32),
            pltpu.VMEM((S, B), f32),
            pltpu.VMEM((S, B), f32),
            pltpu.VMEM((B, H), f32),
            pltpu.VMEM((4, B, qdim), f32),
            pltpu.VMEM((B, att), f32),
            pltpu.VMEM((1, B), jnp.int32),
            pltpu.SMEM((1, B), jnp.int32),
            pltpu.SemaphoreType.DMA,
        ],
        compiler_params=pltpu.CompilerParams(
            vmem_limit_bytes=128 * 1024 * 1024),
    )(tin, wdt, w0t, w1t, w2t, wpt, v_row, g)

    steps_t = tour_steps.T
    tour = jnp.concatenate([steps_t, steps_t[:, :1]], axis=1)
    return (tour, lp[0], ent[0])
